# baseline (speedup 1.0000x reference)
import jax
import jax.numpy as jnp
from jax.experimental import pallas as pl
from jax.experimental.pallas import tpu as pltpu

_BN = 512


def _body(feat_ref, prompts_ref, out_ref, carry_ref):
    j = pl.program_id(0)
    plen = prompts_ref.shape[1]
    b = feat_ref.shape[0]
    d = feat_ref.shape[2]

    ft = jnp.swapaxes(feat_ref[...], 0, 1)
    rolled = pltpu.roll(ft, plen, 0)
    out_ref[...] = rolled
    promptsb = jnp.broadcast_to(prompts_ref[0][:, None, :], (plen, b, d))
    out_ref[:plen] = jnp.where(j == 0, promptsb, carry_ref[...])
    carry_ref[...] = rolled[:plen]


def kernel(features, layer_idx, modality_indices, prompts, prompt_keys):
    del layer_idx, modality_indices
    del prompt_keys
    b, n, d = features.shape
    p, plen, _ = prompts.shape
    assert p == 1, "kernel exploits the single-prompt pool structure"
    bn = _BN if n % _BN == 0 else n
    nf = n // bn
    out = pl.pallas_call(
        _body,
        grid=(nf + 1,),
        in_specs=[
            pl.BlockSpec((b, bn, d), lambda j: (0, jnp.minimum(j, nf - 1), 0)),
            pl.BlockSpec((p, plen, d), lambda j: (0, 0, 0)),
        ],
        out_specs=pl.BlockSpec((bn, b, d), lambda j: (j, 0, 0)),
        out_shape=jax.ShapeDtypeStruct((plen + n, b, d), features.dtype),
        scratch_shapes=[
            pltpu.VMEM((plen, b, d), jnp.float32),
        ],
    )(features, prompts)
    return jnp.swapaxes(out, 0, 1)
